# baseline (speedup 1.0000x reference)
import functools

import jax
import jax.numpy as jnp
from jax import lax
from jax.experimental import pallas as pl
from jax.experimental.pallas import tpu as pltpu
from jax.experimental.pallas import tpu_sc as plsc

_BUDGET = 2048
_SEQ = 8192
_R = 64


def _norms_body(k_ref, norms_ref, valid_ref, count_ref):
    i = pl.program_id(0)
    x = k_ref[...]
    sq = jnp.sum(x * x, axis=-1)
    nr = jnp.sqrt(sq)
    nm = jnp.mean(nr, axis=-1)
    nz = jnp.any(x != 0, axis=-1)
    vd = jnp.any(nz, axis=-1)
    norms_ref[...] = nm
    valid_ref[...] = vd.astype(jnp.float32)
    cnt = jnp.sum(vd.astype(jnp.int32))

    @pl.when(i == 0)
    def _():
        count_ref[0, 0] = 0
    count_ref[0, 0] += cnt


def _norms_pass(k4):
    return pl.pallas_call(
        _norms_body,
        grid=(8,),
        in_specs=[pl.BlockSpec((8, 128, 16, 128), lambda i: (i, 0, 0, 0))],
        out_specs=[
            pl.BlockSpec((8, 128), lambda i: (i, 0)),
            pl.BlockSpec((8, 128), lambda i: (i, 0)),
            pl.BlockSpec(memory_space=pltpu.SMEM),
        ],
        out_shape=[
            jax.ShapeDtypeStruct((_R, 128), jnp.float32),
            jax.ShapeDtypeStruct((_R, 128), jnp.float32),
            jax.ShapeDtypeStruct((1, 1), jnp.int32),
        ],
    )(k4)


def _select_body(al_ref, norms_ref, valid_ref, fi_ref, n2_s, dest_s, sel_s):
    al = al_ref[0]
    norms = norms_ref[...]
    valid = valid_ref[...]
    r_i = lax.broadcasted_iota(jnp.int32, (_R, 128), 0)
    c_i = lax.broadcasted_iota(jnp.int32, (_R, 128), 1)
    gidx = r_i * 128 + c_i
    inf = jnp.float32(jnp.inf)
    n2 = jnp.where(valid > 0, norms, inf)
    n2 = jnp.where(gidx == 0, -inf, n2)
    n2 = jnp.where(gidx == al - 1, -inf, n2)
    gidx_f = gidx.astype(jnp.float32)
    n2_s[...] = n2
    lane = lax.broadcasted_iota(jnp.int32, (1, 128), 1)

    def rank_step(jc, acc):
        row = n2_s[pl.ds(jc, 1), :]
        jrow = (jc * 128 + lane).astype(jnp.float32)
        lt = (row[:, None, :] < n2[:, :, None])
        eq = (row[:, None, :] == n2[:, :, None]) & (
            jrow[:, None, :] < gidx_f[:, :, None])
        return acc + jnp.sum(lt.astype(jnp.float32) + eq.astype(jnp.float32),
                             axis=-1)

    rank = lax.fori_loop(0, _R, rank_step, jnp.zeros((_R, 128), jnp.float32))
    sel = (rank < _BUDGET).astype(jnp.float32)

    cA = lax.broadcasted_iota(jnp.int32, (128, 128), 0)
    cB = lax.broadcasted_iota(jnp.int32, (128, 128), 1)
    M = (cA <= cB).astype(jnp.float32)
    cs_in = jax.lax.dot(sel, M, preferred_element_type=jnp.float32)
    row_tot = jnp.sum(sel, axis=1, keepdims=True)
    rA = lax.broadcasted_iota(jnp.int32, (_R, _R), 0)
    rB = lax.broadcasted_iota(jnp.int32, (_R, _R), 1)
    S = (rB < rA).astype(jnp.float32)
    row_pre = jax.lax.dot(S, row_tot, preferred_element_type=jnp.float32)
    dest = cs_in - sel + row_pre
    dest_s[...] = jnp.where(sel > 0, dest, jnp.float32(-1.0))
    sel_s[...] = sel

    pr = lax.broadcasted_iota(jnp.int32, (16, 128), 0)
    pc = lax.broadcasted_iota(jnp.int32, (16, 128), 1)
    P = (pr * 128 + pc).astype(jnp.float32)

    def fi_step(rc, acc):
        d_row = dest_s[pl.ds(rc, 1), :]
        g_row = (rc * 128 + lane).astype(jnp.float32)
        hit = (d_row[:, None, :] == P[:, :, None])
        return acc + jnp.sum(hit.astype(jnp.float32) * g_row[:, None, :],
                             axis=-1)

    fi = lax.fori_loop(0, _R, fi_step, jnp.zeros((16, 128), jnp.float32))
    fi_ref[...] = fi.astype(jnp.int32)


def _select_pass(al, norms, valid):
    return pl.pallas_call(
        _select_body,
        in_specs=[
            pl.BlockSpec(memory_space=pltpu.SMEM),
            pl.BlockSpec((_R, 128), lambda: (0, 0)),
            pl.BlockSpec((_R, 128), lambda: (0, 0)),
        ],
        out_specs=pl.BlockSpec((16, 128), lambda: (0, 0)),
        out_shape=jax.ShapeDtypeStruct((16, 128), jnp.int32),
        scratch_shapes=[
            pltpu.VMEM((_R, 128), jnp.float32),
            pltpu.VMEM((_R, 128), jnp.float32),
            pltpu.VMEM((_R, 128), jnp.float32),
        ],
    )(al, norms, valid)


_D = 2048
_NW = 32
_GPW = _BUDGET // _NW
_ZPW = (_SEQ - _BUDGET) // _NW
_GC = 16
_ZC = 16


def _gather_sc(fi, k2, v2, z16):
    mesh = plsc.VectorSubcoreMesh(core_axis_name="c", subcore_axis_name="s")

    @functools.partial(
        pl.kernel,
        mesh=mesh,
        out_type=[
            jax.ShapeDtypeStruct((_SEQ, _D), jnp.float32),
            jax.ShapeDtypeStruct((_SEQ, _D), jnp.float32),
        ],
        scratch_types=[
            pltpu.VMEM((_GPW,), jnp.int32),
            pltpu.VMEM((_GC, _D), jnp.float32),
            pltpu.VMEM((_GC, _D), jnp.float32),
            pltpu.VMEM((_ZC, _D), jnp.float32),
            pltpu.SemaphoreType.DMA,
            pltpu.SemaphoreType.DMA,
            pltpu.SemaphoreType.DMA,
            pltpu.SemaphoreType.DMA,
            pltpu.SemaphoreType.DMA,
        ],
    )
    def body(fi_hbm, k_hbm, v_hbm, z_hbm, ko_hbm, vo_hbm,
             idx_v, ra, rb, zbuf, semz, sg0, sg1, ss0, ss1):
        wid = lax.axis_index("s") * 2 + lax.axis_index("c")
        gbase = wid * _GPW
        pltpu.sync_copy(fi_hbm.at[pl.ds(gbase, _GPW)], idx_v)
        pltpu.sync_copy(z_hbm, zbuf)
        zbase = _BUDGET + wid * _ZPW
        zcps = []
        for dst in (ko_hbm, vo_hbm):
            for t in range(_ZPW // _ZC):
                zcps.append(pltpu.async_copy(
                    zbuf, dst.at[pl.ds(zbase + t * _ZC, _ZC)], semz))
        bufs = (ra, rb)
        gsem = (sg0, sg1)
        ssem = (ss0, ss1)
        chunks = []
        for src, dst in ((k_hbm, ko_hbm), (v_hbm, vo_hbm)):
            for c in range(_GPW // _GC):
                chunks.append((src, dst, c * _GC))
        n = len(chunks)

        def start_gather(i):
            src, _, off = chunks[i]
            return pltpu.async_copy(
                src.at[idx_v.at[pl.ds(off, _GC)]], bufs[i % 2], gsem[i % 2])

        g = [None] * n
        s = [None] * n
        g[0] = start_gather(0)
        g[1] = start_gather(1)
        for i in range(n):
            _, dst, off = chunks[i]
            g[i].wait()
            s[i] = pltpu.async_copy(
                bufs[i % 2], dst.at[pl.ds(gbase + off, _GC)], ssem[i % 2])
            if i + 2 < n:
                s[i].wait()
                g[i + 2] = start_gather(i + 2)
        s[n - 2].wait()
        s[n - 1].wait()
        for cp in zcps:
            cp.wait()

    return body(fi, k2, v2, z16)


def kernel(q, k, v):
    seq = k.shape[0]
    k4 = k.reshape(_R, 128, 16, 128)
    norms, valid, al2 = _norms_pass(k4)
    al = al2.reshape((1,))
    actual_len = al[0]

    def do_compress():
        fi2 = _select_pass(al, norms, valid)
        fi = fi2.reshape(_BUDGET)
        k2 = k.reshape(_SEQ, _D)
        v2 = v.reshape(_SEQ, _D)
        z16 = jnp.zeros((_ZC, _D), jnp.float32)
        ko, vo = _gather_sc(fi, k2, v2, z16)
        kp = ko.reshape(k.shape)
        vp = vo.reshape(v.shape)
        return (kp, vp, jnp.array(_BUDGET, jnp.int32),
                actual_len.astype(jnp.int32))

    def do_nothing():
        return (k, v, actual_len.astype(jnp.int32),
                actual_len.astype(jnp.int32))

    return lax.cond(actual_len > _BUDGET, do_compress, do_nothing)


# speedup vs baseline: 2.0148x; 2.0148x over previous
import functools

import jax
import jax.numpy as jnp
from jax import lax
from jax.experimental import pallas as pl
from jax.experimental.pallas import tpu as pltpu
from jax.experimental.pallas import tpu_sc as plsc

_BUDGET = 2048
_SEQ = 8192
_R = 64


def _norms_body(k_ref, norms_ref, valid_ref, count_ref):
    i = pl.program_id(0)
    x = k_ref[...]
    sq = jnp.sum(x * x, axis=-1)
    nr = jnp.sqrt(sq)
    nm = jnp.mean(nr, axis=-1)
    nz = jnp.any(x != 0, axis=-1)
    vd = jnp.any(nz, axis=-1)
    norms_ref[...] = nm
    valid_ref[...] = vd.astype(jnp.float32)
    cnt = jnp.sum(vd.astype(jnp.int32))

    @pl.when(i == 0)
    def _():
        count_ref[0, 0] = 0
    count_ref[0, 0] += cnt


def _norms_pass(k4):
    return pl.pallas_call(
        _norms_body,
        grid=(8,),
        in_specs=[pl.BlockSpec((8, 128, 16, 128), lambda i: (i, 0, 0, 0))],
        out_specs=[
            pl.BlockSpec((8, 128), lambda i: (i, 0)),
            pl.BlockSpec((8, 128), lambda i: (i, 0)),
            pl.BlockSpec(memory_space=pltpu.SMEM),
        ],
        out_shape=[
            jax.ShapeDtypeStruct((_R, 128), jnp.float32),
            jax.ShapeDtypeStruct((_R, 128), jnp.float32),
            jax.ShapeDtypeStruct((1, 1), jnp.int32),
        ],
    )(k4)


def _select_body(al_ref, norms_ref, valid_ref, fi_ref, dest_s):
    al = al_ref[0]
    norms = norms_ref[...]
    valid = valid_ref[...]
    r_i = lax.broadcasted_iota(jnp.int32, (_R, 128), 0)
    c_i = lax.broadcasted_iota(jnp.int32, (_R, 128), 1)
    gidx = r_i * 128 + c_i
    inf = jnp.float32(jnp.inf)
    n2 = jnp.where(valid > 0, norms, inf)
    n2 = jnp.where(gidx == 0, -inf, n2)
    n2 = jnp.where(gidx == al - 1, -inf, n2)

    b = lax.bitcast_convert_type(n2, jnp.int32)
    ku = lax.bitcast_convert_type(n2, jnp.uint32)
    keys = jnp.where(b < 0, ~ku, ku | jnp.uint32(0x80000000))

    def bs_step(_, carry):
        lo, hi = carry
        mid = lo + ((hi - lo) >> 1)
        c = jnp.sum((keys <= mid).astype(jnp.int32))
        big = c >= _BUDGET
        return (jnp.where(big, lo, mid + 1), jnp.where(big, mid, hi))

    lo0 = jnp.uint32(0)
    hi0 = jnp.uint32(0xFFFFFFFF)
    lo, hi = lax.fori_loop(0, 32, bs_step, (lo0, hi0))
    T = lo
    c_lt = jnp.sum((keys < T).astype(jnp.int32))
    need = (_BUDGET - c_lt).astype(jnp.float32)
    eqm = (keys == T).astype(jnp.float32)
    ltm = (keys < T).astype(jnp.float32)

    cA = lax.broadcasted_iota(jnp.int32, (128, 128), 0)
    cB = lax.broadcasted_iota(jnp.int32, (128, 128), 1)
    M = (cA <= cB).astype(jnp.float32)
    rA = lax.broadcasted_iota(jnp.int32, (_R, _R), 0)
    rB = lax.broadcasted_iota(jnp.int32, (_R, _R), 1)
    S = (rB < rA).astype(jnp.float32)

    def excl_cumsum(x):
        cs_in = jax.lax.dot(x, M, preferred_element_type=jnp.float32)
        row_tot = jnp.sum(x, axis=1, keepdims=True)
        row_pre = jax.lax.dot(S, row_tot, preferred_element_type=jnp.float32)
        return cs_in - x + row_pre

    eq_pre = excl_cumsum(eqm)
    sel = jnp.maximum(ltm, eqm * (eq_pre < need).astype(jnp.float32))
    dest = excl_cumsum(sel)
    gidx_f = gidx.astype(jnp.float32)
    dest_s[...] = jnp.where(sel > 0, dest, jnp.float32(-1.0))

    pr = lax.broadcasted_iota(jnp.int32, (16, 128), 0)
    pc = lax.broadcasted_iota(jnp.int32, (16, 128), 1)
    P = (pr * 128 + pc).astype(jnp.float32)
    lane = lax.broadcasted_iota(jnp.int32, (1, 128), 1)

    def fi_step(rc, acc):
        d_row = dest_s[pl.ds(rc, 1), :]
        g_row = (rc * 128 + lane).astype(jnp.float32)
        hit = (d_row[:, None, :] == P[:, :, None])
        return acc + jnp.sum(hit.astype(jnp.float32) * g_row[:, None, :],
                             axis=-1)

    fi = lax.fori_loop(0, _R, fi_step, jnp.zeros((16, 128), jnp.float32))
    fi_ref[...] = fi.astype(jnp.int32)


def _select_pass(al, norms, valid):
    return pl.pallas_call(
        _select_body,
        in_specs=[
            pl.BlockSpec(memory_space=pltpu.SMEM),
            pl.BlockSpec((_R, 128), lambda: (0, 0)),
            pl.BlockSpec((_R, 128), lambda: (0, 0)),
        ],
        out_specs=pl.BlockSpec((16, 128), lambda: (0, 0)),
        out_shape=jax.ShapeDtypeStruct((16, 128), jnp.int32),
        scratch_shapes=[pltpu.VMEM((_R, 128), jnp.float32)],
    )(al, norms, valid)


_D = 2048
_NW = 32
_GPW = _BUDGET // _NW
_ZPW = (_SEQ - _BUDGET) // _NW
_GC = 16
_ZC = 16


def _gather_sc(fi, k3, v3, z16):
    mesh = plsc.VectorSubcoreMesh(core_axis_name="c", subcore_axis_name="s")
    H, DH = k3.shape[1], k3.shape[2]

    @functools.partial(
        pl.kernel,
        mesh=mesh,
        out_type=[
            jax.ShapeDtypeStruct((_SEQ, H, DH), jnp.float32),
            jax.ShapeDtypeStruct((_SEQ, H, DH), jnp.float32),
        ],
        scratch_types=[
            pltpu.VMEM((_GPW,), jnp.int32),
            pltpu.VMEM((_GC, H, DH), jnp.float32),
            pltpu.VMEM((_GC, H, DH), jnp.float32),
            pltpu.VMEM((_ZC, H, DH), jnp.float32),
            pltpu.SemaphoreType.DMA,
            pltpu.SemaphoreType.DMA,
            pltpu.SemaphoreType.DMA,
            pltpu.SemaphoreType.DMA,
            pltpu.SemaphoreType.DMA,
        ],
    )
    def body(fi_hbm, k_hbm, v_hbm, z_hbm, ko_hbm, vo_hbm,
             idx_v, ra, rb, zbuf, semz, sg0, sg1, ss0, ss1):
        wid = lax.axis_index("s") * 2 + lax.axis_index("c")
        gbase = wid * _GPW
        pltpu.sync_copy(fi_hbm.at[pl.ds(gbase, _GPW)], idx_v)
        pltpu.sync_copy(z_hbm, zbuf)
        zbase = _BUDGET + wid * _ZPW
        zcps = []
        for dst in (ko_hbm, vo_hbm):
            for t in range(_ZPW // _ZC):
                zcps.append(pltpu.async_copy(
                    zbuf, dst.at[pl.ds(zbase + t * _ZC, _ZC)], semz))
        bufs = (ra, rb)
        gsem = (sg0, sg1)
        ssem = (ss0, ss1)
        chunks = []
        for src, dst in ((k_hbm, ko_hbm), (v_hbm, vo_hbm)):
            for c in range(_GPW // _GC):
                chunks.append((src, dst, c * _GC))
        n = len(chunks)

        def start_gather(i):
            src, _, off = chunks[i]
            return pltpu.async_copy(
                src.at[idx_v.at[pl.ds(off, _GC)]],
                bufs[i % 2], gsem[i % 2])

        g = [None] * n
        s = [None] * n
        g[0] = start_gather(0)
        g[1] = start_gather(1)
        for i in range(n):
            _, dst, off = chunks[i]
            g[i].wait()
            s[i] = pltpu.async_copy(
                bufs[i % 2], dst.at[pl.ds(gbase + off, _GC)], ssem[i % 2])
            if i + 2 < n:
                s[i].wait()
                g[i + 2] = start_gather(i + 2)
        s[n - 2].wait()
        s[n - 1].wait()
        for cp in zcps:
            cp.wait()

    return body(fi, k3, v3, z16)


def kernel(q, k, v):
    seq = k.shape[0]
    k4 = k.reshape(_R, 128, 16, 128)
    norms, valid, al2 = _norms_pass(k4)
    al = al2.reshape((1,))
    actual_len = al[0]

    def do_compress():
        fi2 = _select_pass(al, norms, valid)
        fi = fi2.reshape(_BUDGET)
        z16 = jnp.zeros((_ZC,) + k.shape[1:], jnp.float32)
        kp, vp = _gather_sc(fi, k, v, z16)
        return (kp, vp, jnp.array(_BUDGET, jnp.int32),
                actual_len.astype(jnp.int32))

    def do_nothing():
        return (k, v, actual_len.astype(jnp.int32),
                actual_len.astype(jnp.int32))

    return lax.cond(actual_len > _BUDGET, do_compress, do_nothing)
